# baseline (speedup 1.0000x reference)
import functools

import jax
import jax.numpy as jnp
from jax import lax
from jax.experimental import pallas as pl
from jax.experimental.pallas import tpu as pltpu
from jax.experimental.pallas import tpu_sc as plsc

_N = 100000
_E = 3200000
_C = 16
_NC = 2
_NS = 16
_NW = _NC * _NS
_CH = 128
_GRP = 160
_EW = 102400
_EP = _EW * _NW
_NCHUNK = _EW // _CH
_NGRP = _NCHUNK // _GRP
_CHN = (1120, 480)
_BASE = (0, _NS * _CHN[0])
_GRPS = 32
_NGRPS = _NCHUNK // _GRPS
_ROWS2D = _EP // _CH
_NN = 100096
_NRT = _NN // _NS

_mesh = plsc.VectorSubcoreMesh(
    core_axis_name="c", subcore_axis_name="s", num_cores=_NC, num_subcores=_NS)
_sc_params = pltpu.CompilerParams(use_tc_tiling_on_sc=False,
                                 needs_layout_passes=False)

_f32 = jnp.float32


def _worker(c, s):
    return c * _NS + s


def _deg_body(dst2_hbm, zer1_hbm, out_hbm, dstv, onesv, dacc, semg):
    c = lax.axis_index("c")
    s = lax.axis_index("s")
    w = _worker(c, s)
    for i in range(_CH // 16):
        onesv[pl.ds(i * 16, 16)] = jnp.full((16,), 1.0, _f32)

    @pl.when(s == 0)
    def _init():
        pltpu.sync_copy(zer1_hbm, dacc)

    plsc.subcore_barrier()

    def work(row0, ngrp):
        def grp(g, carry):
            pltpu.sync_copy(dst2_hbm.at[pl.ds(row0 + g * _GRP, _GRP)], dstv)

            def ch(j, carry2):
                pltpu.sync_copy(onesv, dacc.at[dstv.at[j]], add=True)
                return carry2

            return lax.fori_loop(0, _GRP, ch, carry)

        lax.fori_loop(0, ngrp, grp, 0)

    for ci in range(_NC):
        @pl.when(c == ci)
        def _run(ci=ci):
            work(_BASE[ci] + s * _CHN[ci], _CHN[ci] // _GRP)

    plsc.subcore_barrier()

    @pl.when(s == 0)
    def _flush():
        pltpu.sync_copy(dacc, out_hbm.at[c])


_deg_call = functools.partial(
    pl.kernel,
    out_type=jax.ShapeDtypeStruct((_NC, _NN), _f32),
    mesh=_mesh,
    compiler_params=_sc_params,
    scratch_types=[
        pltpu.VMEM((_GRP, _CH), jnp.int32),
        pltpu.VMEM((_CH,), _f32),
        pltpu.VMEM_SHARED((_NN,), _f32),
        pltpu.SemaphoreType.DMA,
    ],
)(_deg_body)


def _spmm_body(src2_hbm, dst2_hbm, u_hbm, zer2_hbm, out_hbm,
               srcv, dstv, rows_a, rows_b, acc, sem_a, sem_b):
    c = lax.axis_index("c")
    s = lax.axis_index("s")
    w = _worker(c, s)
    pltpu.sync_copy(zer2_hbm.at[pl.ds(s * _NRT, _NRT)],
                    acc.at[pl.ds(s * _NRT, _NRT)])
    plsc.subcore_barrier()

    def work(row0, ngrp):
        def grp(g, carry):
            r0 = row0 + g * _GRPS
            pltpu.sync_copy(src2_hbm.at[pl.ds(r0, _GRPS)], srcv)
            pltpu.sync_copy(dst2_hbm.at[pl.ds(r0, _GRPS)], dstv)
            pltpu.async_copy(u_hbm.at[srcv.at[0]], rows_a, sem_a)

            def pair(t, carry2):
                ja = 2 * t
                jb = 2 * t + 1
                pltpu.make_async_copy(u_hbm.at[srcv.at[ja]], rows_a,
                                      sem_a).wait()
                pltpu.async_copy(u_hbm.at[srcv.at[jb]], rows_b, sem_b)
                pltpu.sync_copy(rows_a, acc.at[dstv.at[ja]], add=True)
                pltpu.make_async_copy(u_hbm.at[srcv.at[jb]], rows_b,
                                      sem_b).wait()

                @pl.when(t < _GRPS // 2 - 1)
                def _next():
                    pltpu.async_copy(u_hbm.at[srcv.at[ja + 2]], rows_a, sem_a)

                pltpu.sync_copy(rows_b, acc.at[dstv.at[jb]], add=True)
                return carry2

            return lax.fori_loop(0, _GRPS // 2, pair, carry)

        lax.fori_loop(0, ngrp, grp, 0)

    for ci in range(_NC):
        @pl.when(c == ci)
        def _run(ci=ci):
            work(_BASE[ci] + s * _CHN[ci], _CHN[ci] // _GRPS)

    plsc.subcore_barrier()
    pltpu.sync_copy(acc.at[pl.ds(s * _NRT, _NRT)],
                    out_hbm.at[c, pl.ds(s * _NRT, _NRT)])


_spmm_call = functools.partial(
    pl.kernel,
    out_type=jax.ShapeDtypeStruct((_NC, _NN, _C), _f32),
    mesh=_mesh,
    compiler_params=_sc_params,
    scratch_types=[
        pltpu.VMEM((_GRPS, _CH), jnp.int32),
        pltpu.VMEM((_GRPS, _CH), jnp.int32),
        pltpu.VMEM((_CH, _C), _f32),
        pltpu.VMEM((_CH, _C), _f32),
        pltpu.VMEM_SHARED((_NN, _C), _f32),
        pltpu.SemaphoreType.DMA,
        pltpu.SemaphoreType.DMA,
    ],
)(_spmm_body)


def _dec_body(src2_hbm, dst2_hbm, h_hbm, out_hbm,
              srcv, dstv, hs_a, hd_a, hs_b, hd_b, valv,
              sem_a1, sem_a2, sem_b1, sem_b2):
    c = lax.axis_index("c")
    s = lax.axis_index("s")
    iota16 = lax.iota(jnp.int32, 16)

    perms = [jnp.bitwise_xor(iota16, sh) for sh in (8, 4, 2, 1)]

    def compute(j, hs, hd):
        def blk(eo, carry2):
            vec = jnp.zeros((16,), _f32)
            for el in range(16):
                e = eo * 16 + el
                p = hs[e] * hd[e]
                for pm in perms:
                    p = p + p.at[pm].get(mode="promise_in_bounds")
                vec = jnp.where(iota16 == el, p, vec)
            sg = 1.0 / (1.0 + jnp.exp(-vec))
            valv[pl.ds(j * _CH + eo * 16, 16)] = sg
            return carry2

        lax.fori_loop(0, _CH // 16, blk, 0)

    def work(row0, ngrp):
        ebase = row0 * _CH

        def grp(g, carry):
            r0 = row0 + g * _GRP
            pltpu.sync_copy(src2_hbm.at[pl.ds(r0, _GRP)], srcv)
            pltpu.sync_copy(dst2_hbm.at[pl.ds(r0, _GRP)], dstv)
            pltpu.async_copy(h_hbm.at[srcv.at[0]], hs_a, sem_a1)
            pltpu.async_copy(h_hbm.at[dstv.at[0]], hd_a, sem_a2)

            def pair(t, carry2):
                ja = 2 * t
                jb = 2 * t + 1
                pltpu.make_async_copy(h_hbm.at[srcv.at[ja]], hs_a,
                                      sem_a1).wait()
                pltpu.make_async_copy(h_hbm.at[dstv.at[ja]], hd_a,
                                      sem_a2).wait()
                pltpu.async_copy(h_hbm.at[srcv.at[jb]], hs_b, sem_b1)
                pltpu.async_copy(h_hbm.at[dstv.at[jb]], hd_b, sem_b2)
                compute(ja, hs_a, hd_a)
                pltpu.make_async_copy(h_hbm.at[srcv.at[jb]], hs_b,
                                      sem_b1).wait()
                pltpu.make_async_copy(h_hbm.at[dstv.at[jb]], hd_b,
                                      sem_b2).wait()

                @pl.when(t < _GRP // 2 - 1)
                def _next():
                    pltpu.async_copy(h_hbm.at[srcv.at[ja + 2]], hs_a, sem_a1)
                    pltpu.async_copy(h_hbm.at[dstv.at[ja + 2]], hd_a, sem_a2)

                compute(jb, hs_b, hd_b)
                return carry2

            lax.fori_loop(0, _GRP // 2, pair, carry)
            pltpu.sync_copy(valv, out_hbm.at[pl.ds(ebase + g * _GRP * _CH,
                                                   _GRP * _CH)])
            return carry

        lax.fori_loop(0, ngrp, grp, 0)

    for ci in range(_NC):
        @pl.when(c == ci)
        def _run(ci=ci):
            work(_BASE[ci] + s * _CHN[ci], _CHN[ci] // _GRP)


_dec_call = functools.partial(
    pl.kernel,
    out_type=jax.ShapeDtypeStruct((_EP,), _f32),
    mesh=_mesh,
    compiler_params=_sc_params,
    scratch_types=[
        pltpu.VMEM((_GRP, _CH), jnp.int32),
        pltpu.VMEM((_GRP, _CH), jnp.int32),
        pltpu.VMEM((_CH, _C), _f32),
        pltpu.VMEM((_CH, _C), _f32),
        pltpu.VMEM((_CH, _C), _f32),
        pltpu.VMEM((_CH, _C), _f32),
        pltpu.VMEM((_GRP * _CH,), _f32),
        pltpu.SemaphoreType.DMA,
        pltpu.SemaphoreType.DMA,
        pltpu.SemaphoreType.DMA,
        pltpu.SemaphoreType.DMA,
    ],
)(_dec_body)


_R = 6256


def _stage1_body(dp_ref, z_ref, w1_ref, dinv_ref, u1_ref):
    deg = dp_ref[0] + dp_ref[1] + 1.0
    dinv = lax.rsqrt(deg)
    dinv_ref[...] = dinv
    u1_ref[...] = dinv * jnp.dot(z_ref[...], w1_ref[...],
                                 preferred_element_type=_f32)


def _stage1(degp, z, W1):
    return pl.pallas_call(
        _stage1_body,
        grid=(_NN // _R,),
        in_specs=[
            pl.BlockSpec((_NC, _R, 1), lambda i: (0, i, 0)),
            pl.BlockSpec((_R, _C), lambda i: (i, 0)),
            pl.BlockSpec((_C, _C), lambda i: (0, 0)),
        ],
        out_specs=[
            pl.BlockSpec((_R, 1), lambda i: (i, 0)),
            pl.BlockSpec((_R, _C), lambda i: (i, 0)),
        ],
        out_shape=[
            jax.ShapeDtypeStruct((_NN, 1), _f32),
            jax.ShapeDtypeStruct((_NN, _C), _f32),
        ],
    )(degp, z, W1)


def _stage2_body(yp_ref, u_ref, dinv_ref, b_ref, w_ref, out_ref):
    h = dinv_ref[...] * (yp_ref[0] + yp_ref[1] + u_ref[...]) + b_ref[...]
    out_ref[...] = dinv_ref[...] * jnp.dot(h, w_ref[...],
                                           preferred_element_type=_f32)


def _stage2(yp, u, dinv, b, W):
    return pl.pallas_call(
        _stage2_body,
        grid=(_NN // _R,),
        in_specs=[
            pl.BlockSpec((_NC, _R, _C), lambda i: (0, i, 0)),
            pl.BlockSpec((_R, _C), lambda i: (i, 0)),
            pl.BlockSpec((_R, 1), lambda i: (i, 0)),
            pl.BlockSpec((1, _C), lambda i: (0, 0)),
            pl.BlockSpec((_C, _C), lambda i: (0, 0)),
        ],
        out_specs=pl.BlockSpec((_R, _C), lambda i: (i, 0)),
        out_shape=jax.ShapeDtypeStruct((_NN, _C), _f32),
    )(yp, u, dinv, b, W)


def _stage3_body(yp_ref, u_ref, dinv_ref, b_ref, out_ref):
    out_ref[...] = (dinv_ref[...] * (yp_ref[0] + yp_ref[1] + u_ref[...])
                    + b_ref[...])


def _stage3(yp, u, dinv, b):
    return pl.pallas_call(
        _stage3_body,
        grid=(_NN // _R,),
        in_specs=[
            pl.BlockSpec((_NC, _R, _C), lambda i: (0, i, 0)),
            pl.BlockSpec((_R, _C), lambda i: (i, 0)),
            pl.BlockSpec((_R, 1), lambda i: (i, 0)),
            pl.BlockSpec((1, _C), lambda i: (0, 0)),
        ],
        out_specs=pl.BlockSpec((_R, _C), lambda i: (i, 0)),
        out_shape=jax.ShapeDtypeStruct((_NN, _C), _f32),
    )(yp, u, dinv, b)


def kernel(z, edge_index, W1, b1, W2, b2):
    pad = jnp.full((_EP - _E,), _N, jnp.int32)
    src2 = jnp.concatenate([edge_index[0], pad]).reshape(_ROWS2D, _CH)
    dst2 = jnp.concatenate([edge_index[1], pad]).reshape(_ROWS2D, _CH)
    zp = jnp.pad(z, ((0, _NN - _N), (0, 0)))
    zeros1 = jnp.zeros((_NN,), _f32)
    zeros2 = jnp.zeros((_NN, _C), _f32)

    degp = _deg_call(dst2, zeros1)
    dinv, u1 = _stage1(degp.reshape(_NC, _NN, 1), zp, W1)
    y1p = _spmm_call(src2, dst2, u1, zeros2)
    u2 = _stage2(y1p, u1, dinv, b1.reshape(1, _C), W2)
    y2p = _spmm_call(src2, dst2, u2, zeros2)
    h2 = _stage3(y2p, u2, dinv, b2.reshape(1, _C))
    return _dec_call(src2, dst2, h2)[:_E]


# speedup vs baseline: 1.0454x; 1.0454x over previous
import functools

import jax
import jax.numpy as jnp
from jax import lax
from jax.experimental import pallas as pl
from jax.experimental.pallas import tpu as pltpu
from jax.experimental.pallas import tpu_sc as plsc

_N = 100000
_E = 3200000
_C = 16
_NC = 2
_NS = 16
_NW = _NC * _NS
_CH = 128
_GRP = 160
_EW = 102400
_EP = _EW * _NW
_NCHUNK = _EW // _CH
_NGRP = _NCHUNK // _GRP
_CHN = (960, 640)
_BASE = (0, _NS * _CHN[0])
_GRPS = 32
_NGRPS = _NCHUNK // _GRPS
_ROWS2D = _EP // _CH
_NN = 100096
_NRT = _NN // _NS

_mesh = plsc.VectorSubcoreMesh(
    core_axis_name="c", subcore_axis_name="s", num_cores=_NC, num_subcores=_NS)
_sc_params = pltpu.CompilerParams(use_tc_tiling_on_sc=False,
                                 needs_layout_passes=False)

_f32 = jnp.float32


def _worker(c, s):
    return c * _NS + s


def _deg_body(dst2_hbm, zer1_hbm, out_hbm, dstv, onesv, dacc, semg):
    c = lax.axis_index("c")
    s = lax.axis_index("s")
    w = _worker(c, s)
    for i in range(_CH // 16):
        onesv[pl.ds(i * 16, 16)] = jnp.full((16,), 1.0, _f32)

    @pl.when(s == 0)
    def _init():
        pltpu.sync_copy(zer1_hbm, dacc)

    plsc.subcore_barrier()

    def work(row0, ngrp):
        def grp(g, carry):
            pltpu.sync_copy(dst2_hbm.at[pl.ds(row0 + g * _GRP, _GRP)], dstv)

            def ch(j, carry2):
                pltpu.sync_copy(onesv, dacc.at[dstv.at[j]], add=True)
                return carry2

            return lax.fori_loop(0, _GRP, ch, carry)

        lax.fori_loop(0, ngrp, grp, 0)

    for ci in range(_NC):
        @pl.when(c == ci)
        def _run(ci=ci):
            work(_BASE[ci] + s * _CHN[ci], _CHN[ci] // _GRP)

    plsc.subcore_barrier()

    @pl.when(s == 0)
    def _flush():
        pltpu.sync_copy(dacc, out_hbm.at[c])


_deg_call = functools.partial(
    pl.kernel,
    out_type=jax.ShapeDtypeStruct((_NC, _NN), _f32),
    mesh=_mesh,
    compiler_params=_sc_params,
    scratch_types=[
        pltpu.VMEM((_GRP, _CH), jnp.int32),
        pltpu.VMEM((_CH,), _f32),
        pltpu.VMEM_SHARED((_NN,), _f32),
        pltpu.SemaphoreType.DMA,
    ],
)(_deg_body)


def _spmm_body(src2_hbm, dst2_hbm, u_hbm, zer2_hbm, out_hbm,
               srcv, dstv, rows_a, rows_b, acc, sem_a, sem_b):
    c = lax.axis_index("c")
    s = lax.axis_index("s")
    w = _worker(c, s)
    pltpu.sync_copy(zer2_hbm.at[pl.ds(s * _NRT, _NRT)],
                    acc.at[pl.ds(s * _NRT, _NRT)])
    plsc.subcore_barrier()

    def work(row0, ngrp):
        def grp(g, carry):
            r0 = row0 + g * _GRPS
            pltpu.sync_copy(src2_hbm.at[pl.ds(r0, _GRPS)], srcv)
            pltpu.sync_copy(dst2_hbm.at[pl.ds(r0, _GRPS)], dstv)
            pltpu.async_copy(u_hbm.at[srcv.at[0]], rows_a, sem_a)

            def pair(t, carry2):
                ja = 2 * t
                jb = 2 * t + 1
                pltpu.make_async_copy(u_hbm.at[srcv.at[ja]], rows_a,
                                      sem_a).wait()
                pltpu.async_copy(u_hbm.at[srcv.at[jb]], rows_b, sem_b)
                pltpu.sync_copy(rows_a, acc.at[dstv.at[ja]], add=True)
                pltpu.make_async_copy(u_hbm.at[srcv.at[jb]], rows_b,
                                      sem_b).wait()

                @pl.when(t < _GRPS // 2 - 1)
                def _next():
                    pltpu.async_copy(u_hbm.at[srcv.at[ja + 2]], rows_a, sem_a)

                pltpu.sync_copy(rows_b, acc.at[dstv.at[jb]], add=True)
                return carry2

            return lax.fori_loop(0, _GRPS // 2, pair, carry)

        lax.fori_loop(0, ngrp, grp, 0)

    for ci in range(_NC):
        @pl.when(c == ci)
        def _run(ci=ci):
            work(_BASE[ci] + s * _CHN[ci], _CHN[ci] // _GRPS)

    plsc.subcore_barrier()
    pltpu.sync_copy(acc.at[pl.ds(s * _NRT, _NRT)],
                    out_hbm.at[c, pl.ds(s * _NRT, _NRT)])


_spmm_call = functools.partial(
    pl.kernel,
    out_type=jax.ShapeDtypeStruct((_NC, _NN, _C), _f32),
    mesh=_mesh,
    compiler_params=_sc_params,
    scratch_types=[
        pltpu.VMEM((_GRPS, _CH), jnp.int32),
        pltpu.VMEM((_GRPS, _CH), jnp.int32),
        pltpu.VMEM((_CH, _C), _f32),
        pltpu.VMEM((_CH, _C), _f32),
        pltpu.VMEM_SHARED((_NN, _C), _f32),
        pltpu.SemaphoreType.DMA,
        pltpu.SemaphoreType.DMA,
    ],
)(_spmm_body)


def _dec_body(src2_hbm, dst2_hbm, h_hbm, out_hbm,
              srcv, dstv, hs_a, hd_a, hs_b, hd_b, valv,
              sem_a1, sem_a2, sem_b1, sem_b2):
    c = lax.axis_index("c")
    s = lax.axis_index("s")
    iota16 = lax.iota(jnp.int32, 16)

    perms = [jnp.bitwise_xor(iota16, sh) for sh in (8, 4, 2, 1)]

    def compute(j, hs, hd):
        def blk(eo, carry2):
            vec = jnp.zeros((16,), _f32)
            for el in range(16):
                e = eo * 16 + el
                p = hs[e] * hd[e]
                for pm in perms:
                    p = p + p.at[pm].get(mode="promise_in_bounds")
                vec = jnp.where(iota16 == el, p, vec)
            sg = 1.0 / (1.0 + jnp.exp(-vec))
            valv[pl.ds(j * _CH + eo * 16, 16)] = sg
            return carry2

        lax.fori_loop(0, _CH // 16, blk, 0)

    def work(row0, ngrp):
        ebase = row0 * _CH

        def grp(g, carry):
            r0 = row0 + g * _GRP
            pltpu.sync_copy(src2_hbm.at[pl.ds(r0, _GRP)], srcv)
            pltpu.sync_copy(dst2_hbm.at[pl.ds(r0, _GRP)], dstv)
            pltpu.async_copy(h_hbm.at[srcv.at[0]], hs_a, sem_a1)
            pltpu.async_copy(h_hbm.at[dstv.at[0]], hd_a, sem_a2)

            def pair(t, carry2):
                ja = 2 * t
                jb = 2 * t + 1
                pltpu.make_async_copy(h_hbm.at[srcv.at[ja]], hs_a,
                                      sem_a1).wait()
                pltpu.make_async_copy(h_hbm.at[dstv.at[ja]], hd_a,
                                      sem_a2).wait()
                pltpu.async_copy(h_hbm.at[srcv.at[jb]], hs_b, sem_b1)
                pltpu.async_copy(h_hbm.at[dstv.at[jb]], hd_b, sem_b2)
                compute(ja, hs_a, hd_a)
                pltpu.make_async_copy(h_hbm.at[srcv.at[jb]], hs_b,
                                      sem_b1).wait()
                pltpu.make_async_copy(h_hbm.at[dstv.at[jb]], hd_b,
                                      sem_b2).wait()

                @pl.when(t < _GRP // 2 - 1)
                def _next():
                    pltpu.async_copy(h_hbm.at[srcv.at[ja + 2]], hs_a, sem_a1)
                    pltpu.async_copy(h_hbm.at[dstv.at[ja + 2]], hd_a, sem_a2)

                compute(jb, hs_b, hd_b)
                return carry2

            lax.fori_loop(0, _GRP // 2, pair, carry)
            pltpu.sync_copy(valv, out_hbm.at[pl.ds(ebase + g * _GRP * _CH,
                                                   _GRP * _CH)])
            return carry

        lax.fori_loop(0, ngrp, grp, 0)

    for ci in range(_NC):
        @pl.when(c == ci)
        def _run(ci=ci):
            work(_BASE[ci] + s * _CHN[ci], _CHN[ci] // _GRP)


_dec_call = functools.partial(
    pl.kernel,
    out_type=jax.ShapeDtypeStruct((_EP,), _f32),
    mesh=_mesh,
    compiler_params=_sc_params,
    scratch_types=[
        pltpu.VMEM((_GRP, _CH), jnp.int32),
        pltpu.VMEM((_GRP, _CH), jnp.int32),
        pltpu.VMEM((_CH, _C), _f32),
        pltpu.VMEM((_CH, _C), _f32),
        pltpu.VMEM((_CH, _C), _f32),
        pltpu.VMEM((_CH, _C), _f32),
        pltpu.VMEM((_GRP * _CH,), _f32),
        pltpu.SemaphoreType.DMA,
        pltpu.SemaphoreType.DMA,
        pltpu.SemaphoreType.DMA,
        pltpu.SemaphoreType.DMA,
    ],
)(_dec_body)


_R = 6256


def _stage1_body(dp_ref, z_ref, w1_ref, dinv_ref, u1_ref):
    deg = dp_ref[0] + dp_ref[1] + 1.0
    dinv = lax.rsqrt(deg)
    dinv_ref[...] = dinv
    u1_ref[...] = dinv * jnp.dot(z_ref[...], w1_ref[...],
                                 preferred_element_type=_f32)


def _stage1(degp, z, W1):
    return pl.pallas_call(
        _stage1_body,
        grid=(_NN // _R,),
        in_specs=[
            pl.BlockSpec((_NC, _R, 1), lambda i: (0, i, 0)),
            pl.BlockSpec((_R, _C), lambda i: (i, 0)),
            pl.BlockSpec((_C, _C), lambda i: (0, 0)),
        ],
        out_specs=[
            pl.BlockSpec((_R, 1), lambda i: (i, 0)),
            pl.BlockSpec((_R, _C), lambda i: (i, 0)),
        ],
        out_shape=[
            jax.ShapeDtypeStruct((_NN, 1), _f32),
            jax.ShapeDtypeStruct((_NN, _C), _f32),
        ],
    )(degp, z, W1)


def _stage2_body(yp_ref, u_ref, dinv_ref, b_ref, w_ref, out_ref):
    h = dinv_ref[...] * (yp_ref[0] + yp_ref[1] + u_ref[...]) + b_ref[...]
    out_ref[...] = dinv_ref[...] * jnp.dot(h, w_ref[...],
                                           preferred_element_type=_f32)


def _stage2(yp, u, dinv, b, W):
    return pl.pallas_call(
        _stage2_body,
        grid=(_NN // _R,),
        in_specs=[
            pl.BlockSpec((_NC, _R, _C), lambda i: (0, i, 0)),
            pl.BlockSpec((_R, _C), lambda i: (i, 0)),
            pl.BlockSpec((_R, 1), lambda i: (i, 0)),
            pl.BlockSpec((1, _C), lambda i: (0, 0)),
            pl.BlockSpec((_C, _C), lambda i: (0, 0)),
        ],
        out_specs=pl.BlockSpec((_R, _C), lambda i: (i, 0)),
        out_shape=jax.ShapeDtypeStruct((_NN, _C), _f32),
    )(yp, u, dinv, b, W)


def _stage3_body(yp_ref, u_ref, dinv_ref, b_ref, out_ref):
    out_ref[...] = (dinv_ref[...] * (yp_ref[0] + yp_ref[1] + u_ref[...])
                    + b_ref[...])


def _stage3(yp, u, dinv, b):
    return pl.pallas_call(
        _stage3_body,
        grid=(_NN // _R,),
        in_specs=[
            pl.BlockSpec((_NC, _R, _C), lambda i: (0, i, 0)),
            pl.BlockSpec((_R, _C), lambda i: (i, 0)),
            pl.BlockSpec((_R, 1), lambda i: (i, 0)),
            pl.BlockSpec((1, _C), lambda i: (0, 0)),
        ],
        out_specs=pl.BlockSpec((_R, _C), lambda i: (i, 0)),
        out_shape=jax.ShapeDtypeStruct((_NN, _C), _f32),
    )(yp, u, dinv, b)


def kernel(z, edge_index, W1, b1, W2, b2):
    pad = jnp.full((_EP - _E,), _N, jnp.int32)
    src2 = jnp.concatenate([edge_index[0], pad]).reshape(_ROWS2D, _CH)
    dst2 = jnp.concatenate([edge_index[1], pad]).reshape(_ROWS2D, _CH)
    zp = jnp.pad(z, ((0, _NN - _N), (0, 0)))
    zeros1 = jnp.zeros((_NN,), _f32)
    zeros2 = jnp.zeros((_NN, _C), _f32)

    degp = _deg_call(dst2, zeros1)
    dinv, u1 = _stage1(degp.reshape(_NC, _NN, 1), zp, W1)
    y1p = _spmm_call(src2, dst2, u1, zeros2)
    u2 = _stage2(y1p, u1, dinv, b1.reshape(1, _C), W2)
    y2p = _spmm_call(src2, dst2, u2, zeros2)
    h2 = _stage3(y2p, u2, dinv, b2.reshape(1, _C))
    return _dec_call(src2, dst2, h2)[:_E]


# speedup vs baseline: 1.1618x; 1.1113x over previous
import functools

import jax
import jax.numpy as jnp
from jax import lax
from jax.experimental import pallas as pl
from jax.experimental.pallas import tpu as pltpu
from jax.experimental.pallas import tpu_sc as plsc

_N = 100000
_E = 3200000
_C = 16
_NC = 2
_NS = 16
_NW = _NC * _NS
_CH = 128
_GRP = 160
_EW = 102400
_EP = _EW * _NW
_NCHUNK = _EW // _CH
_NGRP = _NCHUNK // _GRP
_CHN = (960, 640)
_BASE = (0, _NS * _CHN[0])
_CHND = (1120, 480)
_BASED = (0, _NS * _CHND[0])
_GRPS = 32
_NGRPS = _NCHUNK // _GRPS
_ROWS2D = _EP // _CH
_NN = 100096
_NRT = _NN // _NS

_mesh = plsc.VectorSubcoreMesh(
    core_axis_name="c", subcore_axis_name="s", num_cores=_NC, num_subcores=_NS)
_sc_params = pltpu.CompilerParams(use_tc_tiling_on_sc=False,
                                 needs_layout_passes=False)

_f32 = jnp.float32


def _worker(c, s):
    return c * _NS + s


def _deg_body(dst2_hbm, zer1_hbm, out_hbm, dstv, onesv, dacc, semg):
    c = lax.axis_index("c")
    s = lax.axis_index("s")
    w = _worker(c, s)
    for i in range(_CH // 16):
        onesv[pl.ds(i * 16, 16)] = jnp.full((16,), 1.0, _f32)

    @pl.when(s == 0)
    def _init():
        pltpu.sync_copy(zer1_hbm, dacc)

    plsc.subcore_barrier()

    def work(row0, ngrp):
        def grp(g, carry):
            pltpu.sync_copy(dst2_hbm.at[pl.ds(row0 + g * _GRP, _GRP)], dstv)

            def ch(j, carry2):
                pltpu.sync_copy(onesv, dacc.at[dstv.at[j]], add=True)
                return carry2

            return lax.fori_loop(0, _GRP, ch, carry)

        lax.fori_loop(0, ngrp, grp, 0)

    for ci in range(_NC):
        @pl.when(c == ci)
        def _run(ci=ci):
            work(_BASE[ci] + s * _CHN[ci], _CHN[ci] // _GRP)

    plsc.subcore_barrier()

    @pl.when(s == 0)
    def _flush():
        pltpu.sync_copy(dacc, out_hbm.at[c])


_deg_call = functools.partial(
    pl.kernel,
    out_type=jax.ShapeDtypeStruct((_NC, _NN), _f32),
    mesh=_mesh,
    compiler_params=_sc_params,
    scratch_types=[
        pltpu.VMEM((_GRP, _CH), jnp.int32),
        pltpu.VMEM((_CH,), _f32),
        pltpu.VMEM_SHARED((_NN,), _f32),
        pltpu.SemaphoreType.DMA,
    ],
)(_deg_body)


_NBUF = 4


def _spmm_body(src2_hbm, dst2_hbm, u_hbm, zer2_hbm, out_hbm,
               srcv, dstv, r0b, r1b, r2b, r3b, acc,
               g0, g1, g2, g3, s0, s1, s2, s3):
    c = lax.axis_index("c")
    s = lax.axis_index("s")
    rows = (r0b, r1b, r2b, r3b)
    gsem = (g0, g1, g2, g3)
    ssem = (s0, s1, s2, s3)
    pltpu.sync_copy(zer2_hbm.at[pl.ds(s * _NRT, _NRT)],
                    acc.at[pl.ds(s * _NRT, _NRT)])
    plsc.subcore_barrier()

    def work(row0, ngrp):
        def grp(g, carry):
            r0 = row0 + g * _GRPS
            pltpu.sync_copy(src2_hbm.at[pl.ds(r0, _GRPS)], srcv)
            pltpu.sync_copy(dst2_hbm.at[pl.ds(r0, _GRPS)], dstv)
            for x in range(_NBUF):
                pltpu.async_copy(u_hbm.at[srcv.at[x]], rows[x], gsem[x])

            def quad(t, carry2):
                base = _NBUF * t
                for x in range(_NBUF):
                    j = base + x
                    pltpu.make_async_copy(u_hbm.at[srcv.at[j]], rows[x],
                                          gsem[x]).wait()
                    pltpu.async_copy(rows[x], acc.at[dstv.at[j]], ssem[x],
                                     add=True)
                for x in range(_NBUF):
                    j = base + x
                    pltpu.make_async_copy(rows[x], acc.at[dstv.at[j]],
                                          ssem[x]).wait()

                    @pl.when(t < _GRPS // _NBUF - 1)
                    def _next(x=x, j=j):
                        pltpu.async_copy(u_hbm.at[srcv.at[j + _NBUF]],
                                         rows[x], gsem[x])

                return carry2

            return lax.fori_loop(0, _GRPS // _NBUF, quad, carry)

        lax.fori_loop(0, ngrp, grp, 0)

    for ci in range(_NC):
        @pl.when(c == ci)
        def _run(ci=ci):
            work(_BASE[ci] + s * _CHN[ci], _CHN[ci] // _GRPS)

    plsc.subcore_barrier()
    pltpu.sync_copy(acc.at[pl.ds(s * _NRT, _NRT)],
                    out_hbm.at[c, pl.ds(s * _NRT, _NRT)])


_spmm_call = functools.partial(
    pl.kernel,
    out_type=jax.ShapeDtypeStruct((_NC, _NN, _C), _f32),
    mesh=_mesh,
    compiler_params=_sc_params,
    scratch_types=(
        [pltpu.VMEM((_GRPS, _CH), jnp.int32)] * 2
        + [pltpu.VMEM((_CH, _C), _f32)] * _NBUF
        + [pltpu.VMEM_SHARED((_NN, _C), _f32)]
        + [pltpu.SemaphoreType.DMA] * (2 * _NBUF)
    ),
)(_spmm_body)


def _dec_body(src2_hbm, dst2_hbm, h_hbm, out_hbm,
              srcv, dstv, hs_a, hd_a, hs_b, hd_b, valv,
              sem_a1, sem_a2, sem_b1, sem_b2):
    c = lax.axis_index("c")
    s = lax.axis_index("s")
    iota16 = lax.iota(jnp.int32, 16)

    perms = [jnp.bitwise_xor(iota16, sh) for sh in (8, 4, 2, 1)]

    def compute(j, hs, hd):
        def blk(eo, carry2):
            vec = jnp.zeros((16,), _f32)
            for el in range(16):
                e = eo * 16 + el
                p = hs[e] * hd[e]
                for pm in perms:
                    p = p + p.at[pm].get(mode="promise_in_bounds")
                vec = jnp.where(iota16 == el, p, vec)
            sg = 1.0 / (1.0 + jnp.exp(-vec))
            valv[pl.ds(j * _CH + eo * 16, 16)] = sg
            return carry2

        lax.fori_loop(0, _CH // 16, blk, 0)

    def work(row0, ngrp):
        ebase = row0 * _CH

        def grp(g, carry):
            r0 = row0 + g * _GRP
            pltpu.sync_copy(src2_hbm.at[pl.ds(r0, _GRP)], srcv)
            pltpu.sync_copy(dst2_hbm.at[pl.ds(r0, _GRP)], dstv)
            pltpu.async_copy(h_hbm.at[srcv.at[0]], hs_a, sem_a1)
            pltpu.async_copy(h_hbm.at[dstv.at[0]], hd_a, sem_a2)

            def pair(t, carry2):
                ja = 2 * t
                jb = 2 * t + 1
                pltpu.make_async_copy(h_hbm.at[srcv.at[ja]], hs_a,
                                      sem_a1).wait()
                pltpu.make_async_copy(h_hbm.at[dstv.at[ja]], hd_a,
                                      sem_a2).wait()
                pltpu.async_copy(h_hbm.at[srcv.at[jb]], hs_b, sem_b1)
                pltpu.async_copy(h_hbm.at[dstv.at[jb]], hd_b, sem_b2)
                compute(ja, hs_a, hd_a)
                pltpu.make_async_copy(h_hbm.at[srcv.at[jb]], hs_b,
                                      sem_b1).wait()
                pltpu.make_async_copy(h_hbm.at[dstv.at[jb]], hd_b,
                                      sem_b2).wait()

                @pl.when(t < _GRP // 2 - 1)
                def _next():
                    pltpu.async_copy(h_hbm.at[srcv.at[ja + 2]], hs_a, sem_a1)
                    pltpu.async_copy(h_hbm.at[dstv.at[ja + 2]], hd_a, sem_a2)

                compute(jb, hs_b, hd_b)
                return carry2

            lax.fori_loop(0, _GRP // 2, pair, carry)
            pltpu.sync_copy(valv, out_hbm.at[pl.ds(ebase + g * _GRP * _CH,
                                                   _GRP * _CH)])
            return carry

        lax.fori_loop(0, ngrp, grp, 0)

    for ci in range(_NC):
        @pl.when(c == ci)
        def _run(ci=ci):
            work(_BASED[ci] + s * _CHND[ci], _CHND[ci] // _GRP)


_dec_call = functools.partial(
    pl.kernel,
    out_type=jax.ShapeDtypeStruct((_EP,), _f32),
    mesh=_mesh,
    compiler_params=_sc_params,
    scratch_types=[
        pltpu.VMEM((_GRP, _CH), jnp.int32),
        pltpu.VMEM((_GRP, _CH), jnp.int32),
        pltpu.VMEM((_CH, _C), _f32),
        pltpu.VMEM((_CH, _C), _f32),
        pltpu.VMEM((_CH, _C), _f32),
        pltpu.VMEM((_CH, _C), _f32),
        pltpu.VMEM((_GRP * _CH,), _f32),
        pltpu.SemaphoreType.DMA,
        pltpu.SemaphoreType.DMA,
        pltpu.SemaphoreType.DMA,
        pltpu.SemaphoreType.DMA,
    ],
)(_dec_body)


_R = 6256


def _stage1_body(dp_ref, z_ref, w1_ref, dinv_ref, u1_ref):
    deg = dp_ref[0] + dp_ref[1] + 1.0
    dinv = lax.rsqrt(deg)
    dinv_ref[...] = dinv
    u1_ref[...] = dinv * jnp.dot(z_ref[...], w1_ref[...],
                                 preferred_element_type=_f32)


def _stage1(degp, z, W1):
    return pl.pallas_call(
        _stage1_body,
        grid=(_NN // _R,),
        in_specs=[
            pl.BlockSpec((_NC, _R, 1), lambda i: (0, i, 0)),
            pl.BlockSpec((_R, _C), lambda i: (i, 0)),
            pl.BlockSpec((_C, _C), lambda i: (0, 0)),
        ],
        out_specs=[
            pl.BlockSpec((_R, 1), lambda i: (i, 0)),
            pl.BlockSpec((_R, _C), lambda i: (i, 0)),
        ],
        out_shape=[
            jax.ShapeDtypeStruct((_NN, 1), _f32),
            jax.ShapeDtypeStruct((_NN, _C), _f32),
        ],
    )(degp, z, W1)


def _stage2_body(yp_ref, u_ref, dinv_ref, b_ref, w_ref, out_ref):
    h = dinv_ref[...] * (yp_ref[0] + yp_ref[1] + u_ref[...]) + b_ref[...]
    out_ref[...] = dinv_ref[...] * jnp.dot(h, w_ref[...],
                                           preferred_element_type=_f32)


def _stage2(yp, u, dinv, b, W):
    return pl.pallas_call(
        _stage2_body,
        grid=(_NN // _R,),
        in_specs=[
            pl.BlockSpec((_NC, _R, _C), lambda i: (0, i, 0)),
            pl.BlockSpec((_R, _C), lambda i: (i, 0)),
            pl.BlockSpec((_R, 1), lambda i: (i, 0)),
            pl.BlockSpec((1, _C), lambda i: (0, 0)),
            pl.BlockSpec((_C, _C), lambda i: (0, 0)),
        ],
        out_specs=pl.BlockSpec((_R, _C), lambda i: (i, 0)),
        out_shape=jax.ShapeDtypeStruct((_NN, _C), _f32),
    )(yp, u, dinv, b, W)


def _stage3_body(yp_ref, u_ref, dinv_ref, b_ref, out_ref):
    out_ref[...] = (dinv_ref[...] * (yp_ref[0] + yp_ref[1] + u_ref[...])
                    + b_ref[...])


def _stage3(yp, u, dinv, b):
    return pl.pallas_call(
        _stage3_body,
        grid=(_NN // _R,),
        in_specs=[
            pl.BlockSpec((_NC, _R, _C), lambda i: (0, i, 0)),
            pl.BlockSpec((_R, _C), lambda i: (i, 0)),
            pl.BlockSpec((_R, 1), lambda i: (i, 0)),
            pl.BlockSpec((1, _C), lambda i: (0, 0)),
        ],
        out_specs=pl.BlockSpec((_R, _C), lambda i: (i, 0)),
        out_shape=jax.ShapeDtypeStruct((_NN, _C), _f32),
    )(yp, u, dinv, b)


def kernel(z, edge_index, W1, b1, W2, b2):
    pad = jnp.full((_EP - _E,), _N, jnp.int32)
    src2 = jnp.concatenate([edge_index[0], pad]).reshape(_ROWS2D, _CH)
    dst2 = jnp.concatenate([edge_index[1], pad]).reshape(_ROWS2D, _CH)
    zp = jnp.pad(z, ((0, _NN - _N), (0, 0)))
    zeros1 = jnp.zeros((_NN,), _f32)
    zeros2 = jnp.zeros((_NN, _C), _f32)

    degp = _deg_call(dst2, zeros1)
    dinv, u1 = _stage1(degp.reshape(_NC, _NN, 1), zp, W1)
    y1p = _spmm_call(src2, dst2, u1, zeros2)
    u2 = _stage2(y1p, u1, dinv, b1.reshape(1, _C), W2)
    y2p = _spmm_call(src2, dst2, u2, zeros2)
    h2 = _stage3(y2p, u2, dinv, b2.reshape(1, _C))
    return _dec_call(src2, dst2, h2)[:_E]


# speedup vs baseline: 1.2068x; 1.0388x over previous
import functools

import jax
import jax.numpy as jnp
from jax import lax
from jax.experimental import pallas as pl
from jax.experimental.pallas import tpu as pltpu
from jax.experimental.pallas import tpu_sc as plsc

_N = 100000
_E = 3200000
_C = 16
_NC = 2
_NS = 16
_NW = _NC * _NS
_CH = 128
_GRP = 160
_EW = 102400
_EP = _EW * _NW
_NCHUNK = _EW // _CH
_NGRP = _NCHUNK // _GRP
_CHN = (960, 640)
_BASE = (0, _NS * _CHN[0])
_CHND = (1120, 480)
_BASED = (0, _NS * _CHND[0])
_GRPS = 32
_NGRPS = _NCHUNK // _GRPS
_ROWS2D = _EP // _CH
_NN = 100096
_NRT = _NN // _NS

_mesh = plsc.VectorSubcoreMesh(
    core_axis_name="c", subcore_axis_name="s", num_cores=_NC, num_subcores=_NS)
_sc_params = pltpu.CompilerParams(use_tc_tiling_on_sc=False,
                                 needs_layout_passes=False)

_f32 = jnp.float32


def _worker(c, s):
    return c * _NS + s


def _deg_body(dst2_hbm, zer1_hbm, out_hbm, dstv, onesv, dacc, semg):
    c = lax.axis_index("c")
    s = lax.axis_index("s")
    w = _worker(c, s)
    for i in range(_CH // 16):
        onesv[pl.ds(i * 16, 16)] = jnp.full((16,), 1.0, _f32)

    @pl.when(s == 0)
    def _init():
        pltpu.sync_copy(zer1_hbm, dacc)

    plsc.subcore_barrier()

    def work(row0, ngrp):
        def grp(g, carry):
            pltpu.sync_copy(dst2_hbm.at[pl.ds(row0 + g * _GRP, _GRP)], dstv)

            def ch(j, carry2):
                pltpu.sync_copy(onesv, dacc.at[dstv.at[j]], add=True)
                return carry2

            return lax.fori_loop(0, _GRP, ch, carry)

        lax.fori_loop(0, ngrp, grp, 0)

    for ci in range(_NC):
        @pl.when(c == ci)
        def _run(ci=ci):
            work(_BASE[ci] + s * _CHN[ci], _CHN[ci] // _GRP)

    plsc.subcore_barrier()

    @pl.when(s == 0)
    def _flush():
        pltpu.sync_copy(dacc, out_hbm.at[c])


_deg_call = functools.partial(
    pl.kernel,
    out_type=jax.ShapeDtypeStruct((_NC, _NN), _f32),
    mesh=_mesh,
    compiler_params=_sc_params,
    scratch_types=[
        pltpu.VMEM((_GRP, _CH), jnp.int32),
        pltpu.VMEM((_CH,), _f32),
        pltpu.VMEM_SHARED((_NN,), _f32),
        pltpu.SemaphoreType.DMA,
    ],
)(_deg_body)


_NBUF = 4


def _spmm_body(src2_hbm, dst2_hbm, u_hbm, zer2_hbm, out_hbm,
               srcv, dstv, r0b, r1b, r2b, r3b, acc,
               g0, g1, g2, g3, s0, s1, s2, s3):
    c = lax.axis_index("c")
    s = lax.axis_index("s")
    rows = (r0b, r1b, r2b, r3b)
    gsem = (g0, g1, g2, g3)
    ssem = (s0, s1, s2, s3)
    pltpu.sync_copy(zer2_hbm.at[pl.ds(s * _NRT, _NRT)],
                    acc.at[pl.ds(s * _NRT, _NRT)])
    plsc.subcore_barrier()

    def work(row0, ngrp):
        def grp(g, carry):
            r0 = row0 + g * _GRPS
            pltpu.sync_copy(src2_hbm.at[pl.ds(r0, _GRPS)], srcv)
            pltpu.sync_copy(dst2_hbm.at[pl.ds(r0, _GRPS)], dstv)
            for x in range(_NBUF):
                pltpu.async_copy(u_hbm.at[srcv.at[x]], rows[x], gsem[x])

            def quad(t, carry2):
                base = _NBUF * t
                for x in range(_NBUF):
                    j = base + x
                    pltpu.make_async_copy(u_hbm.at[srcv.at[j]], rows[x],
                                          gsem[x]).wait()
                    pltpu.async_copy(rows[x], acc.at[dstv.at[j]], ssem[x],
                                     add=True)
                for x in range(_NBUF):
                    j = base + x
                    pltpu.make_async_copy(rows[x], acc.at[dstv.at[j]],
                                          ssem[x]).wait()

                    @pl.when(t < _GRPS // _NBUF - 1)
                    def _next(x=x, j=j):
                        pltpu.async_copy(u_hbm.at[srcv.at[j + _NBUF]],
                                         rows[x], gsem[x])

                return carry2

            return lax.fori_loop(0, _GRPS // _NBUF, quad, carry)

        lax.fori_loop(0, ngrp, grp, 0)

    for ci in range(_NC):
        @pl.when(c == ci)
        def _run(ci=ci):
            work(_BASE[ci] + s * _CHN[ci], _CHN[ci] // _GRPS)

    plsc.subcore_barrier()
    pltpu.sync_copy(acc.at[pl.ds(s * _NRT, _NRT)],
                    out_hbm.at[c, pl.ds(s * _NRT, _NRT)])


_spmm_call = functools.partial(
    pl.kernel,
    out_type=jax.ShapeDtypeStruct((_NC, _NN, _C), _f32),
    mesh=_mesh,
    compiler_params=_sc_params,
    scratch_types=(
        [pltpu.VMEM((_GRPS, _CH), jnp.int32)] * 2
        + [pltpu.VMEM((_CH, _C), _f32)] * _NBUF
        + [pltpu.VMEM_SHARED((_NN, _C), _f32)]
        + [pltpu.SemaphoreType.DMA] * (2 * _NBUF)
    ),
)(_spmm_body)


def _dec_body(src2_hbm, dst2_hbm, h_hbm, out_hbm,
              srcv, dstv, hs0, hs1, hs2, hs3, hd0, hd1, hd2, hd3, valv,
              gs0, gs1, gs2, gs3, gd0, gd1, gd2, gd3):
    c = lax.axis_index("c")
    s = lax.axis_index("s")
    hs = (hs0, hs1, hs2, hs3)
    hd = (hd0, hd1, hd2, hd3)
    gs = (gs0, gs1, gs2, gs3)
    gd = (gd0, gd1, gd2, gd3)
    iota16 = lax.iota(jnp.int32, 16)

    perms = [jnp.bitwise_xor(iota16, sh) for sh in (8, 4, 2, 1)]

    def compute(j, hsx, hdx):
        def blk(eo, carry2):
            vec = jnp.zeros((16,), _f32)
            for el in range(16):
                e = eo * 16 + el
                p = hsx[e] * hdx[e]
                for pm in perms:
                    p = p + p.at[pm].get(mode="promise_in_bounds")
                vec = jnp.where(iota16 == el, p, vec)
            sg = 1.0 / (1.0 + jnp.exp(-vec))
            valv[pl.ds(j * _CH + eo * 16, 16)] = sg
            return carry2

        lax.fori_loop(0, _CH // 16, blk, 0)

    def work(row0, ngrp):
        ebase = row0 * _CH

        def grp(g, carry):
            r0 = row0 + g * _GRP
            pltpu.sync_copy(src2_hbm.at[pl.ds(r0, _GRP)], srcv)
            pltpu.sync_copy(dst2_hbm.at[pl.ds(r0, _GRP)], dstv)
            for x in range(_NBUF):
                pltpu.async_copy(h_hbm.at[srcv.at[x]], hs[x], gs[x])
                pltpu.async_copy(h_hbm.at[dstv.at[x]], hd[x], gd[x])

            def quad(t, carry2):
                base = _NBUF * t
                for x in range(_NBUF):
                    j = base + x
                    pltpu.make_async_copy(h_hbm.at[srcv.at[j]], hs[x],
                                          gs[x]).wait()
                    pltpu.make_async_copy(h_hbm.at[dstv.at[j]], hd[x],
                                          gd[x]).wait()
                    compute(j, hs[x], hd[x])

                    @pl.when(t < _GRP // _NBUF - 1)
                    def _next(x=x, j=j):
                        pltpu.async_copy(h_hbm.at[srcv.at[j + _NBUF]],
                                         hs[x], gs[x])
                        pltpu.async_copy(h_hbm.at[dstv.at[j + _NBUF]],
                                         hd[x], gd[x])

                return carry2

            lax.fori_loop(0, _GRP // _NBUF, quad, carry)
            pltpu.sync_copy(valv, out_hbm.at[pl.ds(ebase + g * _GRP * _CH,
                                                   _GRP * _CH)])
            return carry

        lax.fori_loop(0, ngrp, grp, 0)

    for ci in range(_NC):
        @pl.when(c == ci)
        def _run(ci=ci):
            work(_BASED[ci] + s * _CHND[ci], _CHND[ci] // _GRP)


_dec_call = functools.partial(
    pl.kernel,
    out_type=jax.ShapeDtypeStruct((_EP,), _f32),
    mesh=_mesh,
    compiler_params=_sc_params,
    scratch_types=(
        [pltpu.VMEM((_GRP, _CH), jnp.int32)] * 2
        + [pltpu.VMEM((_CH, _C), _f32)] * (2 * _NBUF)
        + [pltpu.VMEM((_GRP * _CH,), _f32)]
        + [pltpu.SemaphoreType.DMA] * (2 * _NBUF)
    ),
)(_dec_body)


_R = 6256


def _stage1_body(dp_ref, z_ref, w1_ref, dinv_ref, u1_ref):
    deg = dp_ref[0] + dp_ref[1] + 1.0
    dinv = lax.rsqrt(deg)
    dinv_ref[...] = dinv
    u1_ref[...] = dinv * jnp.dot(z_ref[...], w1_ref[...],
                                 preferred_element_type=_f32)


def _stage1(degp, z, W1):
    return pl.pallas_call(
        _stage1_body,
        grid=(_NN // _R,),
        in_specs=[
            pl.BlockSpec((_NC, _R, 1), lambda i: (0, i, 0)),
            pl.BlockSpec((_R, _C), lambda i: (i, 0)),
            pl.BlockSpec((_C, _C), lambda i: (0, 0)),
        ],
        out_specs=[
            pl.BlockSpec((_R, 1), lambda i: (i, 0)),
            pl.BlockSpec((_R, _C), lambda i: (i, 0)),
        ],
        out_shape=[
            jax.ShapeDtypeStruct((_NN, 1), _f32),
            jax.ShapeDtypeStruct((_NN, _C), _f32),
        ],
    )(degp, z, W1)


def _stage2_body(yp_ref, u_ref, dinv_ref, b_ref, w_ref, out_ref):
    h = dinv_ref[...] * (yp_ref[0] + yp_ref[1] + u_ref[...]) + b_ref[...]
    out_ref[...] = dinv_ref[...] * jnp.dot(h, w_ref[...],
                                           preferred_element_type=_f32)


def _stage2(yp, u, dinv, b, W):
    return pl.pallas_call(
        _stage2_body,
        grid=(_NN // _R,),
        in_specs=[
            pl.BlockSpec((_NC, _R, _C), lambda i: (0, i, 0)),
            pl.BlockSpec((_R, _C), lambda i: (i, 0)),
            pl.BlockSpec((_R, 1), lambda i: (i, 0)),
            pl.BlockSpec((1, _C), lambda i: (0, 0)),
            pl.BlockSpec((_C, _C), lambda i: (0, 0)),
        ],
        out_specs=pl.BlockSpec((_R, _C), lambda i: (i, 0)),
        out_shape=jax.ShapeDtypeStruct((_NN, _C), _f32),
    )(yp, u, dinv, b, W)


def _stage3_body(yp_ref, u_ref, dinv_ref, b_ref, out_ref):
    out_ref[...] = (dinv_ref[...] * (yp_ref[0] + yp_ref[1] + u_ref[...])
                    + b_ref[...])


def _stage3(yp, u, dinv, b):
    return pl.pallas_call(
        _stage3_body,
        grid=(_NN // _R,),
        in_specs=[
            pl.BlockSpec((_NC, _R, _C), lambda i: (0, i, 0)),
            pl.BlockSpec((_R, _C), lambda i: (i, 0)),
            pl.BlockSpec((_R, 1), lambda i: (i, 0)),
            pl.BlockSpec((1, _C), lambda i: (0, 0)),
        ],
        out_specs=pl.BlockSpec((_R, _C), lambda i: (i, 0)),
        out_shape=jax.ShapeDtypeStruct((_NN, _C), _f32),
    )(yp, u, dinv, b)


def kernel(z, edge_index, W1, b1, W2, b2):
    pad = jnp.full((_EP - _E,), _N, jnp.int32)
    src2 = jnp.concatenate([edge_index[0], pad]).reshape(_ROWS2D, _CH)
    dst2 = jnp.concatenate([edge_index[1], pad]).reshape(_ROWS2D, _CH)
    zp = jnp.pad(z, ((0, _NN - _N), (0, 0)))
    zeros1 = jnp.zeros((_NN,), _f32)
    zeros2 = jnp.zeros((_NN, _C), _f32)

    degp = _deg_call(dst2, zeros1)
    dinv, u1 = _stage1(degp.reshape(_NC, _NN, 1), zp, W1)
    y1p = _spmm_call(src2, dst2, u1, zeros2)
    u2 = _stage2(y1p, u1, dinv, b1.reshape(1, _C), W2)
    y2p = _spmm_call(src2, dst2, u2, zeros2)
    h2 = _stage3(y2p, u2, dinv, b2.reshape(1, _C))
    return _dec_call(src2, dst2, h2)[:_E]
